# Initial kernel scaffold; baseline (speedup 1.0000x reference)
import jax
import jax.numpy as jnp
from jax.experimental import pallas as pl


def kernel(x, edge_index, gW1, gb1, gW2, gb2, gW3, gb3, gW4, gb4, lW1, lb1, lW2, lb2, dlW1, dlb1, dlW2, dlb2, dgW1, dgb1, dgW2, dgb2, dgW3, dgb3, dgW4, dgb4):
    raise NotImplementedError("write your pallas kernel here")


# speedup vs baseline: 18.5452x; 18.5452x over previous
import functools

import jax
import jax.numpy as jnp
from jax import lax
from jax.experimental import pallas as pl
from jax.experimental.pallas import tpu as pltpu
from jax.experimental.pallas import tpu_sc as plsc

N = 10000
NP = 10240
NC, NS = 2, 16
NW = NC * NS
CH = 128
BN = 1280


@functools.partial(jax.jit, static_argnums=(4,))
def _sc_propagate(y, srcr, dstr, zeros, nchunks):
    F = y.shape[1]
    rows_per_tile = NP // NS
    mesh = plsc.VectorSubcoreMesh(core_axis_name="c", subcore_axis_name="s")

    def body(y_hbm, srcr_hbm, dstr_hbm, zeros_hbm, out_hbm,
             src_v, dst_v, rows_v, acc, sem):
        c = lax.axis_index("c")
        s = lax.axis_index("s")
        wid = s * NC + c
        row0 = s * rows_per_tile
        pltpu.sync_copy(zeros_hbm.at[pl.ds(row0, rows_per_tile)],
                        acc.at[pl.ds(row0, rows_per_tile)])
        pltpu.sync_copy(srcr_hbm.at[wid], src_v)
        pltpu.sync_copy(dstr_hbm.at[wid], dst_v)
        plsc.subcore_barrier()

        def step(j, carry):
            pltpu.async_copy(y_hbm.at[src_v.at[j]], rows_v, sem).wait()
            pltpu.sync_copy(rows_v, acc.at[dst_v.at[j]], add=True)
            return carry

        lax.fori_loop(0, nchunks, step, 0, unroll=False)
        plsc.subcore_barrier()
        pltpu.sync_copy(acc.at[pl.ds(row0, rows_per_tile)],
                        out_hbm.at[c, pl.ds(row0, rows_per_tile)])

    run = pl.kernel(
        body,
        out_type=jax.ShapeDtypeStruct((NC, NP, F), jnp.float32),
        mesh=mesh,
        scratch_types=[
            pltpu.VMEM((nchunks, CH), jnp.int32),
            pltpu.VMEM((nchunks, CH), jnp.int32),
            pltpu.VMEM((CH, F), jnp.float32),
            pltpu.VMEM_SHARED((NP, F), jnp.float32),
            pltpu.SemaphoreType.DMA,
        ],
        compiler_params=pltpu.CompilerParams(use_tc_tiling_on_sc=False),
    )
    return run(y, srcr, dstr, zeros)


def _tc_call(fn, out_widths, *args):
    in_specs = []
    for a in args:
        if a.ndim == 2 and a.shape[0] == NP:
            in_specs.append(
                pl.BlockSpec((BN, a.shape[1]), lambda i: (i, 0)))
        else:
            nd = a.ndim
            in_specs.append(
                pl.BlockSpec(a.shape, lambda i, _nd=nd: (0,) * _nd))
    out_specs = [pl.BlockSpec((BN, w), lambda i: (i, 0)) for w in out_widths]
    out_shape = [jax.ShapeDtypeStruct((NP, w), jnp.float32)
                 for w in out_widths]
    outs = pl.pallas_call(
        fn,
        grid=(NP // BN,),
        in_specs=in_specs,
        out_specs=out_specs,
        out_shape=out_shape,
    )(*args)
    return outs


def _mm(a, w):
    return jnp.dot(a, w, preferred_element_type=jnp.float32)


def kernel(x, edge_index, gW1, gb1, gW2, gb2, gW3, gb3, gW4, gb4,
           lW1, lb1, lW2, lb2, dlW1, dlb1, dlW2, dlb2,
           dgW1, dgb1, dgW2, dgb2, dgW3, dgb3, dgW4, dgb4):
    f32 = jnp.float32
    E = edge_index.shape[1]
    T = -(-E // (NW * CH))
    EP = NW * T * CH
    pad = EP - E
    pad_idx = (N + jnp.arange(pad, dtype=jnp.int32) % 16)
    srcr = jnp.concatenate(
        [edge_index[0].astype(jnp.int32), pad_idx]).reshape(NW, T, CH)
    dstr = jnp.concatenate(
        [edge_index[1].astype(jnp.int32), pad_idx]).reshape(NW, T, CH)
    xp = jnp.concatenate([x.astype(f32), jnp.zeros((NP - N, 128), f32)])
    ones8 = jnp.ones((NP, 8), f32)
    z8 = jnp.zeros((NP, 8), f32)

    def bias(b):
        return b.reshape(1, -1).astype(f32)

    dp = _sc_propagate(ones8, srcr, dstr, z8, T)

    def s0(dp0, dp1, x_r, dinv_o, y1_o):
        dinv = lax.rsqrt(dp0[:, 0:1] + dp1[:, 0:1] + 1.0)
        dinv_o[...] = dinv
        y1_o[...] = dinv * x_r[...]

    dinv, y1 = _tc_call(s0, [1, 128], dp[0], dp[1], xp)

    def prop(y, F):
        zz = jnp.zeros((NP, F), f32)
        return _sc_propagate(y, srcr, dstr, zz, T)

    p = prop(y1, 128)

    def s1(p0, p1, y, dv, W1, b1, W2, y2_o):
        t = dv[...] * (p0[...] + p1[...] + y[...])
        h = jax.nn.relu(_mm(t, W1[...]) + b1[...])
        y2_o[...] = dv[...] * _mm(h, W2[...])

    (y2,) = _tc_call(s1, [80], p[0], p[1], y1, dinv, gW1, bias(gb1), gW2)

    p = prop(y2, 80)

    def s2(p0, p1, y, dv, b2, W3, y3_o):
        h = jax.nn.relu(dv[...] * (p0[...] + p1[...] + y[...]) + b2[...])
        y3_o[...] = dv[...] * _mm(h, W3[...])

    (y3,) = _tc_call(s2, [40], p[0], p[1], y2, dinv, bias(gb2), gW3)

    p = prop(y3, 40)

    def s3(p0, p1, y, dv, b3, W4, y4_o):
        h = jax.nn.relu(dv[...] * (p0[...] + p1[...] + y[...]) + b3[...])
        y4_o[...] = dv[...] * _mm(h, W4[...])

    gW4p = jnp.concatenate([gW4.astype(f32), jnp.zeros((40, 12), f32)], 1)
    (y4,) = _tc_call(s3, [32], p[0], p[1], y3, dinv, bias(gb3), gW4p)

    p = prop(y4, 32)

    def s4(p0, p1, y, dv, b4, W_l1, b_l1, W_l2, b_l2,
           W_d1, b_d1, W_d2, b_d2, y5_o):
        h = jax.nn.relu(dv[...] * (p0[...] + p1[...] + y[...]) + b4[...])
        h = jax.nn.relu(_mm(h, W_l1[...]) + b_l1[...])
        h = _mm(h, W_l2[...]) + b_l2[...]
        h = jax.nn.relu(_mm(h, W_d1[...]) + b_d1[...])
        h = jax.nn.relu(_mm(h, W_d2[...]) + b_d2[...])
        y5_o[...] = dv[...] * h

    gb4p = jnp.concatenate([gb4.astype(f32), jnp.zeros((12,), f32)])
    lW1p = jnp.concatenate([lW1.astype(f32), jnp.zeros((12, 10), f32)], 0)
    dlW2p = jnp.concatenate([dlW2.astype(f32), jnp.zeros((10, 12), f32)], 1)
    dlb2p = jnp.concatenate([dlb2.astype(f32), jnp.zeros((12,), f32)])
    (y5,) = _tc_call(s4, [32], p[0], p[1], y4, dinv, bias(gb4p),
                     lW1p, bias(lb1), lW2, bias(lb2),
                     dlW1, bias(dlb1), dlW2p, bias(dlb2p))

    p = prop(y5, 32)

    def s5(p0, p1, y, dv, W, b, y6_o):
        t = dv[...] * (p0[...] + p1[...] + y[...])
        y6_o[...] = dv[...] * jax.nn.relu(_mm(t, W[...]) + b[...])

    dgW1p = jnp.concatenate([dgW1.astype(f32), jnp.zeros((12, 40), f32)], 0)
    (y6,) = _tc_call(s5, [40], p[0], p[1], y5, dinv, dgW1p, bias(dgb1))

    p = prop(y6, 40)

    def s6(p0, p1, y, dv, W, b, y7_o):
        t = dv[...] * (p0[...] + p1[...] + y[...])
        y7_o[...] = dv[...] * jax.nn.relu(_mm(t, W[...]) + b[...])

    (y7,) = _tc_call(s6, [80], p[0], p[1], y6, dinv, dgW2, bias(dgb2))

    p = prop(y7, 80)

    def s7(p0, p1, y, dv, W3_, b3_, W4_, y8_o):
        t = dv[...] * (p0[...] + p1[...] + y[...])
        h = jax.nn.relu(_mm(t, W3_[...]) + b3_[...])
        y8_o[...] = dv[...] * _mm(h, W4_[...])

    dgW4p = jnp.concatenate([dgW4.astype(f32), jnp.zeros((160, 5), f32)], 1)
    (y8,) = _tc_call(s7, [8], p[0], p[1], y7, dinv, dgW3, bias(dgb3), dgW4p)

    p = prop(y8, 8)
    dgb4p = jnp.concatenate([dgb4.astype(f32), jnp.zeros((5,), f32)])

    def s8(p0, p1, y, dv, b, out_o):
        out_o[...] = jnp.tanh(
            dv[...] * (p0[...] + p1[...] + y[...]) + b[...])

    (out,) = _tc_call(s8, [8], p[0], p[1], y8, dinv, bias(dgb4p))
    return out[:N, :3]


# speedup vs baseline: 25.1916x; 1.3584x over previous
import functools

import jax
import jax.numpy as jnp
from jax import lax
from jax.experimental import pallas as pl
from jax.experimental.pallas import tpu as pltpu
from jax.experimental.pallas import tpu_sc as plsc

N = 10000
NP = 10240
NC, NS = 2, 16
NW = NC * NS
CH = 128
BN = 1280


@functools.partial(jax.jit, static_argnums=(4,))
def _sc_propagate(y, srcr, dstr, zeros, nchunks):
    F = y.shape[1]
    rows_per_tile = NP // NS
    NB = 2 if F > 80 else 4
    SB = 16
    mesh = plsc.VectorSubcoreMesh(core_axis_name="c", subcore_axis_name="s")

    def body(y_hbm, srcr_hbm, dstr_hbm, zeros_hbm, out_hbm,
             src_v, dst_v, rows, sems, acc):
        c = lax.axis_index("c")
        s = lax.axis_index("s")
        wid = s * NC + c
        row0 = s * rows_per_tile
        pltpu.sync_copy(zeros_hbm.at[pl.ds(row0, rows_per_tile)],
                        acc.at[pl.ds(row0, rows_per_tile)])
        plsc.subcore_barrier()

        def outer(q, carry):
            pltpu.sync_copy(srcr_hbm.at[wid, pl.ds(q * SB, SB)], src_v)
            pltpu.sync_copy(dstr_hbm.at[wid, pl.ds(q * SB, SB)], dst_v)
            for b in range(NB - 1):
                pltpu.async_copy(y_hbm.at[src_v.at[b]], rows.at[b],
                                 sems.at[b])
            for k in range(SB):
                b = k % NB
                nxt = k + NB - 1
                if nxt < SB:
                    bi = nxt % NB
                    pltpu.async_copy(y_hbm.at[src_v.at[nxt]], rows.at[bi],
                                     sems.at[bi])
                pltpu.make_async_copy(y_hbm.at[pl.ds(0, CH)], rows.at[b],
                                      sems.at[b]).wait()
                pltpu.sync_copy(rows.at[b], acc.at[dst_v.at[k]], add=True)
            return carry

        lax.fori_loop(0, nchunks // SB, outer, 0, unroll=False)
        plsc.subcore_barrier()
        pltpu.sync_copy(acc.at[pl.ds(row0, rows_per_tile)],
                        out_hbm.at[c, pl.ds(row0, rows_per_tile)])

    run = pl.kernel(
        body,
        out_type=jax.ShapeDtypeStruct((NC, NP, F), jnp.float32),
        mesh=mesh,
        scratch_types=[
            pltpu.VMEM((SB, CH), jnp.int32),
            pltpu.VMEM((SB, CH), jnp.int32),
            pltpu.VMEM((NB, CH, F), jnp.float32),
            pltpu.SemaphoreType.DMA((NB,)),
            pltpu.VMEM_SHARED((NP, F), jnp.float32),
        ],
        compiler_params=pltpu.CompilerParams(use_tc_tiling_on_sc=False),
    )
    return run(y, srcr, dstr, zeros)


def _tc_call(fn, out_widths, *args):
    in_specs = []
    for a in args:
        if a.ndim == 2 and a.shape[0] == NP:
            in_specs.append(
                pl.BlockSpec((BN, a.shape[1]), lambda i: (i, 0)))
        else:
            nd = a.ndim
            in_specs.append(
                pl.BlockSpec(a.shape, lambda i, _nd=nd: (0,) * _nd))
    out_specs = [pl.BlockSpec((BN, w), lambda i: (i, 0)) for w in out_widths]
    out_shape = [jax.ShapeDtypeStruct((NP, w), jnp.float32)
                 for w in out_widths]
    outs = pl.pallas_call(
        fn,
        grid=(NP // BN,),
        in_specs=in_specs,
        out_specs=out_specs,
        out_shape=out_shape,
    )(*args)
    return outs


def _mm(a, w):
    return jnp.dot(a, w, preferred_element_type=jnp.float32)


def kernel(x, edge_index, gW1, gb1, gW2, gb2, gW3, gb3, gW4, gb4,
           lW1, lb1, lW2, lb2, dlW1, dlb1, dlW2, dlb2,
           dgW1, dgb1, dgW2, dgb2, dgW3, dgb3, dgW4, dgb4):
    f32 = jnp.float32
    E = edge_index.shape[1]
    T = -(-E // (NW * CH))
    T = -(-T // 16) * 16
    EP = NW * T * CH
    pad = EP - E
    pad_idx = (N + jnp.arange(pad, dtype=jnp.int32) % 16)
    srcr = jnp.concatenate(
        [edge_index[0].astype(jnp.int32), pad_idx]).reshape(NW, T, CH)
    dstr = jnp.concatenate(
        [edge_index[1].astype(jnp.int32), pad_idx]).reshape(NW, T, CH)
    xp = jnp.concatenate([x.astype(f32), jnp.zeros((NP - N, 128), f32)])
    ones8 = jnp.ones((NP, 8), f32)
    z8 = jnp.zeros((NP, 8), f32)

    def bias(b):
        return b.reshape(1, -1).astype(f32)

    dp = _sc_propagate(ones8, srcr, dstr, z8, T)

    def s0(dp0, dp1, x_r, dinv_o, y1_o):
        dinv = lax.rsqrt(dp0[:, 0:1] + dp1[:, 0:1] + 1.0)
        dinv_o[...] = dinv
        y1_o[...] = dinv * x_r[...]

    dinv, y1 = _tc_call(s0, [1, 128], dp[0], dp[1], xp)

    def prop(y, F):
        zz = jnp.zeros((NP, F), f32)
        return _sc_propagate(y, srcr, dstr, zz, T)

    p = prop(y1, 128)

    def s1(p0, p1, y, dv, W1, b1, W2, y2_o):
        t = dv[...] * (p0[...] + p1[...] + y[...])
        h = jax.nn.relu(_mm(t, W1[...]) + b1[...])
        y2_o[...] = dv[...] * _mm(h, W2[...])

    (y2,) = _tc_call(s1, [80], p[0], p[1], y1, dinv, gW1, bias(gb1), gW2)

    p = prop(y2, 80)

    def s2(p0, p1, y, dv, b2, W3, y3_o):
        h = jax.nn.relu(dv[...] * (p0[...] + p1[...] + y[...]) + b2[...])
        y3_o[...] = dv[...] * _mm(h, W3[...])

    (y3,) = _tc_call(s2, [40], p[0], p[1], y2, dinv, bias(gb2), gW3)

    p = prop(y3, 40)

    def s3(p0, p1, y, dv, b3, W4, y4_o):
        h = jax.nn.relu(dv[...] * (p0[...] + p1[...] + y[...]) + b3[...])
        y4_o[...] = dv[...] * _mm(h, W4[...])

    gW4p = jnp.concatenate([gW4.astype(f32), jnp.zeros((40, 12), f32)], 1)
    (y4,) = _tc_call(s3, [32], p[0], p[1], y3, dinv, bias(gb3), gW4p)

    p = prop(y4, 32)

    def s4(p0, p1, y, dv, b4, W_l1, b_l1, W_l2, b_l2,
           W_d1, b_d1, W_d2, b_d2, y5_o):
        h = jax.nn.relu(dv[...] * (p0[...] + p1[...] + y[...]) + b4[...])
        h = jax.nn.relu(_mm(h, W_l1[...]) + b_l1[...])
        h = _mm(h, W_l2[...]) + b_l2[...]
        h = jax.nn.relu(_mm(h, W_d1[...]) + b_d1[...])
        h = jax.nn.relu(_mm(h, W_d2[...]) + b_d2[...])
        y5_o[...] = dv[...] * h

    gb4p = jnp.concatenate([gb4.astype(f32), jnp.zeros((12,), f32)])
    lW1p = jnp.concatenate([lW1.astype(f32), jnp.zeros((12, 10), f32)], 0)
    dlW2p = jnp.concatenate([dlW2.astype(f32), jnp.zeros((10, 12), f32)], 1)
    dlb2p = jnp.concatenate([dlb2.astype(f32), jnp.zeros((12,), f32)])
    (y5,) = _tc_call(s4, [32], p[0], p[1], y4, dinv, bias(gb4p),
                     lW1p, bias(lb1), lW2, bias(lb2),
                     dlW1, bias(dlb1), dlW2p, bias(dlb2p))

    p = prop(y5, 32)

    def s5(p0, p1, y, dv, W, b, y6_o):
        t = dv[...] * (p0[...] + p1[...] + y[...])
        y6_o[...] = dv[...] * jax.nn.relu(_mm(t, W[...]) + b[...])

    dgW1p = jnp.concatenate([dgW1.astype(f32), jnp.zeros((12, 40), f32)], 0)
    (y6,) = _tc_call(s5, [40], p[0], p[1], y5, dinv, dgW1p, bias(dgb1))

    p = prop(y6, 40)

    def s6(p0, p1, y, dv, W, b, y7_o):
        t = dv[...] * (p0[...] + p1[...] + y[...])
        y7_o[...] = dv[...] * jax.nn.relu(_mm(t, W[...]) + b[...])

    (y7,) = _tc_call(s6, [80], p[0], p[1], y6, dinv, dgW2, bias(dgb2))

    p = prop(y7, 80)

    def s7(p0, p1, y, dv, W3_, b3_, W4_, y8_o):
        t = dv[...] * (p0[...] + p1[...] + y[...])
        h = jax.nn.relu(_mm(t, W3_[...]) + b3_[...])
        y8_o[...] = dv[...] * _mm(h, W4_[...])

    dgW4p = jnp.concatenate([dgW4.astype(f32), jnp.zeros((160, 5), f32)], 1)
    (y8,) = _tc_call(s7, [8], p[0], p[1], y7, dinv, dgW3, bias(dgb3), dgW4p)

    p = prop(y8, 8)
    dgb4p = jnp.concatenate([dgb4.astype(f32), jnp.zeros((5,), f32)])

    def s8(p0, p1, y, dv, b, out_o):
        out_o[...] = jnp.tanh(
            dv[...] * (p0[...] + p1[...] + y[...]) + b[...])

    (out,) = _tc_call(s8, [8], p[0], p[1], y8, dinv, bias(dgb4p))
    return out[:N, :3]


# speedup vs baseline: 26.5648x; 1.0545x over previous
import functools

import jax
import jax.numpy as jnp
from jax import lax
from jax.experimental import pallas as pl
from jax.experimental.pallas import tpu as pltpu
from jax.experimental.pallas import tpu_sc as plsc

N = 10000
NP = 10240
NC, NS = 2, 16
NW = NC * NS
CH = 128
BN = 1280


@functools.partial(jax.jit, static_argnums=(4, 5))
def _sc_propagate(y, srcr, dstr, zeros, nchunks, constant_rows=False):
    F = y.shape[1]
    rows_per_tile = NP // NS
    staged = False
    NB = 1 if constant_rows else (2 if F > 80 else (4 if F > 48 else 8))
    SB = 16
    mesh = plsc.VectorSubcoreMesh(core_axis_name="c", subcore_axis_name="s")

    def body(y_hbm, srcr_hbm, dstr_hbm, zeros_hbm, out_hbm,
             src_v, dst_v, rows, sems, acc, tbl):
        c = lax.axis_index("c")
        s = lax.axis_index("s")
        wid = s * NC + c
        row0 = s * rows_per_tile
        pltpu.sync_copy(zeros_hbm.at[pl.ds(row0, rows_per_tile)],
                        acc.at[pl.ds(row0, rows_per_tile)])
        if staged:
            pltpu.sync_copy(y_hbm.at[pl.ds(row0, rows_per_tile)],
                            tbl.at[pl.ds(row0, rows_per_tile)])
        if constant_rows:
            pltpu.sync_copy(y_hbm.at[pl.ds(0, CH)], rows.at[0])
        plsc.subcore_barrier()
        src = tbl if staged else y_hbm

        def outer(q, carry):
            if not constant_rows:
                pltpu.sync_copy(srcr_hbm.at[wid, pl.ds(q * SB, SB)], src_v)
            pltpu.sync_copy(dstr_hbm.at[wid, pl.ds(q * SB, SB)], dst_v)
            if constant_rows:
                for k in range(SB):
                    pltpu.sync_copy(rows.at[0], acc.at[dst_v.at[k]],
                                    add=True)
                return carry
            for b in range(NB - 1):
                pltpu.async_copy(src.at[src_v.at[b]], rows.at[b],
                                 sems.at[b])
            for k in range(SB):
                b = k % NB
                nxt = k + NB - 1
                if nxt < SB:
                    bi = nxt % NB
                    pltpu.async_copy(src.at[src_v.at[nxt]], rows.at[bi],
                                     sems.at[bi])
                pltpu.make_async_copy(y_hbm.at[pl.ds(0, CH)], rows.at[b],
                                      sems.at[b]).wait()
                pltpu.sync_copy(rows.at[b], acc.at[dst_v.at[k]], add=True)
            return carry

        lax.fori_loop(0, nchunks // SB, outer, 0, unroll=False)
        plsc.subcore_barrier()
        pltpu.sync_copy(acc.at[pl.ds(row0, rows_per_tile)],
                        out_hbm.at[c, pl.ds(row0, rows_per_tile)])

    scratch = [
        pltpu.VMEM((SB, CH), jnp.int32),
        pltpu.VMEM((SB, CH), jnp.int32),
        pltpu.VMEM((NB, CH, F), jnp.float32),
        pltpu.SemaphoreType.DMA((NB,)),
        pltpu.VMEM_SHARED((NP, F), jnp.float32),
        pltpu.VMEM_SHARED((NP, F) if staged else (8, 8), jnp.float32),
    ]
    run = pl.kernel(
        body,
        out_type=jax.ShapeDtypeStruct((NC, NP, F), jnp.float32),
        mesh=mesh,
        scratch_types=scratch,
        compiler_params=pltpu.CompilerParams(use_tc_tiling_on_sc=False),
    )
    return run(y, srcr, dstr, zeros)


def _tc_call(fn, out_widths, *args):
    in_specs = []
    for a in args:
        if a.ndim == 2 and a.shape[0] == NP:
            in_specs.append(
                pl.BlockSpec((BN, a.shape[1]), lambda i: (i, 0)))
        else:
            nd = a.ndim
            in_specs.append(
                pl.BlockSpec(a.shape, lambda i, _nd=nd: (0,) * _nd))
    out_specs = [pl.BlockSpec((BN, w), lambda i: (i, 0)) for w in out_widths]
    out_shape = [jax.ShapeDtypeStruct((NP, w), jnp.float32)
                 for w in out_widths]
    outs = pl.pallas_call(
        fn,
        grid=(NP // BN,),
        in_specs=in_specs,
        out_specs=out_specs,
        out_shape=out_shape,
    )(*args)
    return outs


def _mm(a, w):
    return jnp.dot(a, w, preferred_element_type=jnp.float32)


def kernel(x, edge_index, gW1, gb1, gW2, gb2, gW3, gb3, gW4, gb4,
           lW1, lb1, lW2, lb2, dlW1, dlb1, dlW2, dlb2,
           dgW1, dgb1, dgW2, dgb2, dgW3, dgb3, dgW4, dgb4):
    f32 = jnp.float32
    E = edge_index.shape[1]
    T = -(-E // (NW * CH))
    T = -(-T // 16) * 16
    EP = NW * T * CH
    pad = EP - E
    pad_idx = (N + jnp.arange(pad, dtype=jnp.int32) % 16)
    srcr = jnp.concatenate(
        [edge_index[0].astype(jnp.int32), pad_idx]).reshape(NW, T, CH)
    dstr = jnp.concatenate(
        [edge_index[1].astype(jnp.int32), pad_idx]).reshape(NW, T, CH)
    xp = jnp.concatenate([x.astype(f32), jnp.zeros((NP - N, 128), f32)])
    ones8 = jnp.ones((NP, 8), f32)
    z8 = jnp.zeros((NP, 8), f32)

    def bias(b):
        return b.reshape(1, -1).astype(f32)

    dp = _sc_propagate(ones8, srcr, dstr, z8, T, True)

    def s0(dp0, dp1, x_r, dinv_o, y1_o):
        dinv = lax.rsqrt(dp0[:, 0:1] + dp1[:, 0:1] + 1.0)
        dinv_o[...] = dinv
        y1_o[...] = dinv * x_r[...]

    dinv, y1 = _tc_call(s0, [1, 128], dp[0], dp[1], xp)

    def prop(y, F):
        zz = jnp.zeros((NP, F), f32)
        return _sc_propagate(y, srcr, dstr, zz, T)

    p = prop(y1, 128)

    def s1(p0, p1, y, dv, W1, b1, W2, y2_o):
        t = dv[...] * (p0[...] + p1[...] + y[...])
        h = jax.nn.relu(_mm(t, W1[...]) + b1[...])
        y2_o[...] = dv[...] * _mm(h, W2[...])

    (y2,) = _tc_call(s1, [80], p[0], p[1], y1, dinv, gW1, bias(gb1), gW2)

    p = prop(y2, 80)

    def s2(p0, p1, y, dv, b2, W3, y3_o):
        h = jax.nn.relu(dv[...] * (p0[...] + p1[...] + y[...]) + b2[...])
        y3_o[...] = dv[...] * _mm(h, W3[...])

    (y3,) = _tc_call(s2, [40], p[0], p[1], y2, dinv, bias(gb2), gW3)

    p = prop(y3, 40)

    def s3(p0, p1, y, dv, b3, W4, y4_o):
        h = jax.nn.relu(dv[...] * (p0[...] + p1[...] + y[...]) + b3[...])
        y4_o[...] = dv[...] * _mm(h, W4[...])

    gW4p = jnp.concatenate([gW4.astype(f32), jnp.zeros((40, 12), f32)], 1)
    (y4,) = _tc_call(s3, [32], p[0], p[1], y3, dinv, bias(gb3), gW4p)

    p = prop(y4, 32)

    def s4(p0, p1, y, dv, b4, W_l1, b_l1, W_l2, b_l2,
           W_d1, b_d1, W_d2, b_d2, y5_o):
        h = jax.nn.relu(dv[...] * (p0[...] + p1[...] + y[...]) + b4[...])
        h = jax.nn.relu(_mm(h, W_l1[...]) + b_l1[...])
        h = _mm(h, W_l2[...]) + b_l2[...]
        h = jax.nn.relu(_mm(h, W_d1[...]) + b_d1[...])
        h = jax.nn.relu(_mm(h, W_d2[...]) + b_d2[...])
        y5_o[...] = dv[...] * h

    gb4p = jnp.concatenate([gb4.astype(f32), jnp.zeros((12,), f32)])
    lW1p = jnp.concatenate([lW1.astype(f32), jnp.zeros((12, 10), f32)], 0)
    dlW2p = jnp.concatenate([dlW2.astype(f32), jnp.zeros((10, 12), f32)], 1)
    dlb2p = jnp.concatenate([dlb2.astype(f32), jnp.zeros((12,), f32)])
    (y5,) = _tc_call(s4, [32], p[0], p[1], y4, dinv, bias(gb4p),
                     lW1p, bias(lb1), lW2, bias(lb2),
                     dlW1, bias(dlb1), dlW2p, bias(dlb2p))

    p = prop(y5, 32)

    def s5(p0, p1, y, dv, W, b, y6_o):
        t = dv[...] * (p0[...] + p1[...] + y[...])
        y6_o[...] = dv[...] * jax.nn.relu(_mm(t, W[...]) + b[...])

    dgW1p = jnp.concatenate([dgW1.astype(f32), jnp.zeros((12, 40), f32)], 0)
    (y6,) = _tc_call(s5, [40], p[0], p[1], y5, dinv, dgW1p, bias(dgb1))

    p = prop(y6, 40)

    def s6(p0, p1, y, dv, W, b, y7_o):
        t = dv[...] * (p0[...] + p1[...] + y[...])
        y7_o[...] = dv[...] * jax.nn.relu(_mm(t, W[...]) + b[...])

    (y7,) = _tc_call(s6, [80], p[0], p[1], y6, dinv, dgW2, bias(dgb2))

    p = prop(y7, 80)

    def s7(p0, p1, y, dv, W3_, b3_, W4_, y8_o):
        t = dv[...] * (p0[...] + p1[...] + y[...])
        h = jax.nn.relu(_mm(t, W3_[...]) + b3_[...])
        y8_o[...] = dv[...] * _mm(h, W4_[...])

    dgW4p = jnp.concatenate([dgW4.astype(f32), jnp.zeros((160, 5), f32)], 1)
    (y8,) = _tc_call(s7, [8], p[0], p[1], y7, dinv, dgW3, bias(dgb3), dgW4p)

    p = prop(y8, 8)
    dgb4p = jnp.concatenate([dgb4.astype(f32), jnp.zeros((5,), f32)])

    def s8(p0, p1, y, dv, b, out_o):
        out_o[...] = jnp.tanh(
            dv[...] * (p0[...] + p1[...] + y[...]) + b[...])

    (out,) = _tc_call(s8, [8], p[0], p[1], y8, dinv, bias(dgb4p))
    return out[:N, :3]


# speedup vs baseline: 29.4087x; 1.1071x over previous
import functools

import jax
import jax.numpy as jnp
from jax import lax
from jax.experimental import pallas as pl
from jax.experimental.pallas import tpu as pltpu
from jax.experimental.pallas import tpu_sc as plsc

N = 10000
NP = 10240
NC, NS = 2, 16
NW = NC * NS
CH = 128
BN = 1280


_CFG = {128: (160, 64, 8), 80: (392, 26, 13), 40: (1000, 10, 10),
        32: (1000, 10, 10), 8: (5000, 2, 2)}


@functools.partial(jax.jit, static_argnums=(4, 5, 6))
def _sc_propagate(y, srcr, dstr, zeros, T, QB, constant_rows=False):
    F = y.shape[1]
    BCH = srcr.shape[2]
    rows_per_tile = NP // NS
    NB = 1 if constant_rows else 2
    mesh = plsc.VectorSubcoreMesh(core_axis_name="c", subcore_axis_name="s")

    def body(y_hbm, srcr_hbm, dstr_hbm, zeros_hbm, out_hbm,
             src_v, dst_v, rows, sems, acc):
        c = lax.axis_index("c")
        s = lax.axis_index("s")
        wid = s * NC + c
        row0 = s * rows_per_tile
        pltpu.sync_copy(zeros_hbm.at[pl.ds(row0, rows_per_tile)],
                        acc.at[pl.ds(row0, rows_per_tile)])
        if constant_rows:
            pltpu.sync_copy(y_hbm.at[pl.ds(0, BCH)], rows.at[0])
        plsc.subcore_barrier()

        def outer(q, carry):
            if not constant_rows:
                pltpu.sync_copy(srcr_hbm.at[wid, pl.ds(q * QB, QB)], src_v)
            pltpu.sync_copy(dstr_hbm.at[wid, pl.ds(q * QB, QB)], dst_v)
            if constant_rows:
                for k in range(QB):
                    pltpu.sync_copy(rows.at[0], acc.at[dst_v.at[k]],
                                    add=True)
                return carry
            for b in range(NB - 1):
                pltpu.async_copy(y_hbm.at[src_v.at[b]], rows.at[b],
                                 sems.at[b])
            for k in range(QB):
                b = k % NB
                nxt = k + NB - 1
                if nxt < QB:
                    bi = nxt % NB
                    pltpu.async_copy(y_hbm.at[src_v.at[nxt]], rows.at[bi],
                                     sems.at[bi])
                pltpu.make_async_copy(y_hbm.at[pl.ds(0, BCH)], rows.at[b],
                                      sems.at[b]).wait()
                pltpu.sync_copy(rows.at[b], acc.at[dst_v.at[k]], add=True)
            return carry

        lax.fori_loop(0, T // QB, outer, 0, unroll=False)
        plsc.subcore_barrier()
        pltpu.sync_copy(acc.at[pl.ds(row0, rows_per_tile)],
                        out_hbm.at[c, pl.ds(row0, rows_per_tile)])

    scratch = [
        pltpu.VMEM((QB, BCH), jnp.int32),
        pltpu.VMEM((QB, BCH), jnp.int32),
        pltpu.VMEM((NB, BCH, F), jnp.float32),
        pltpu.SemaphoreType.DMA((NB,)),
        pltpu.VMEM_SHARED((NP, F), jnp.float32),
    ]
    run = pl.kernel(
        body,
        out_type=jax.ShapeDtypeStruct((NC, NP, F), jnp.float32),
        mesh=mesh,
        scratch_types=scratch,
        compiler_params=pltpu.CompilerParams(use_tc_tiling_on_sc=False),
    )
    return run(y, srcr, dstr, zeros)


def _tc_call(fn, out_widths, *args):
    in_specs = []
    for a in args:
        if a.ndim == 2 and a.shape[0] == NP:
            in_specs.append(
                pl.BlockSpec((BN, a.shape[1]), lambda i: (i, 0)))
        else:
            nd = a.ndim
            in_specs.append(
                pl.BlockSpec(a.shape, lambda i, _nd=nd: (0,) * _nd))
    out_specs = [pl.BlockSpec((BN, w), lambda i: (i, 0)) for w in out_widths]
    out_shape = [jax.ShapeDtypeStruct((NP, w), jnp.float32)
                 for w in out_widths]
    outs = pl.pallas_call(
        fn,
        grid=(NP // BN,),
        in_specs=in_specs,
        out_specs=out_specs,
        out_shape=out_shape,
    )(*args)
    return outs


def _mm(a, w):
    return jnp.dot(a, w, preferred_element_type=jnp.float32)


def kernel(x, edge_index, gW1, gb1, gW2, gb2, gW3, gb3, gW4, gb4,
           lW1, lb1, lW2, lb2, dlW1, dlb1, dlW2, dlb2,
           dgW1, dgb1, dgW2, dgb2, dgW3, dgb3, dgW4, dgb4):
    f32 = jnp.float32
    E = edge_index.shape[1]
    idx_arrays = {}
    for bch, t, _qb in _CFG.values():
        if bch in idx_arrays:
            continue
        ep = NW * t * bch
        pad_idx = (N + jnp.arange(ep - E, dtype=jnp.int32) % 16)
        idx_arrays[bch] = (
            jnp.concatenate(
                [edge_index[0].astype(jnp.int32), pad_idx]).reshape(
                    NW, t, bch),
            jnp.concatenate(
                [edge_index[1].astype(jnp.int32), pad_idx]).reshape(
                    NW, t, bch))
    xp = jnp.concatenate([x.astype(f32), jnp.zeros((NP - N, 128), f32)])
    ones8 = jnp.ones((NP, 8), f32)
    z8 = jnp.zeros((NP, 8), f32)

    def bias(b):
        return b.reshape(1, -1).astype(f32)

    bch8, t8, qb8 = _CFG[8]
    dp = _sc_propagate(ones8, idx_arrays[bch8][0], idx_arrays[bch8][1],
                       z8, t8, qb8, True)

    def s0(dp0, dp1, x_r, dinv_o, y1_o):
        dinv = lax.rsqrt(dp0[:, 0:1] + dp1[:, 0:1] + 1.0)
        dinv_o[...] = dinv
        y1_o[...] = dinv * x_r[...]

    dinv, y1 = _tc_call(s0, [1, 128], dp[0], dp[1], xp)

    def prop(y, F):
        bch, t, qb = _CFG[F]
        zz = jnp.zeros((NP, F), f32)
        return _sc_propagate(y, idx_arrays[bch][0], idx_arrays[bch][1],
                             zz, t, qb, False)

    p = prop(y1, 128)

    def s1(p0, p1, y, dv, W1, b1, W2, y2_o):
        t = dv[...] * (p0[...] + p1[...] + y[...])
        h = jax.nn.relu(_mm(t, W1[...]) + b1[...])
        y2_o[...] = dv[...] * _mm(h, W2[...])

    (y2,) = _tc_call(s1, [80], p[0], p[1], y1, dinv, gW1, bias(gb1), gW2)

    p = prop(y2, 80)

    def s2(p0, p1, y, dv, b2, W3, y3_o):
        h = jax.nn.relu(dv[...] * (p0[...] + p1[...] + y[...]) + b2[...])
        y3_o[...] = dv[...] * _mm(h, W3[...])

    (y3,) = _tc_call(s2, [40], p[0], p[1], y2, dinv, bias(gb2), gW3)

    p = prop(y3, 40)

    def s3(p0, p1, y, dv, b3, W4, y4_o):
        h = jax.nn.relu(dv[...] * (p0[...] + p1[...] + y[...]) + b3[...])
        y4_o[...] = dv[...] * _mm(h, W4[...])

    gW4p = jnp.concatenate([gW4.astype(f32), jnp.zeros((40, 12), f32)], 1)
    (y4,) = _tc_call(s3, [32], p[0], p[1], y3, dinv, bias(gb3), gW4p)

    p = prop(y4, 32)

    def s4(p0, p1, y, dv, b4, W_l1, b_l1, W_l2, b_l2,
           W_d1, b_d1, W_d2, b_d2, y5_o):
        h = jax.nn.relu(dv[...] * (p0[...] + p1[...] + y[...]) + b4[...])
        h = jax.nn.relu(_mm(h, W_l1[...]) + b_l1[...])
        h = _mm(h, W_l2[...]) + b_l2[...]
        h = jax.nn.relu(_mm(h, W_d1[...]) + b_d1[...])
        h = jax.nn.relu(_mm(h, W_d2[...]) + b_d2[...])
        y5_o[...] = dv[...] * h

    gb4p = jnp.concatenate([gb4.astype(f32), jnp.zeros((12,), f32)])
    lW1p = jnp.concatenate([lW1.astype(f32), jnp.zeros((12, 10), f32)], 0)
    dlW2p = jnp.concatenate([dlW2.astype(f32), jnp.zeros((10, 12), f32)], 1)
    dlb2p = jnp.concatenate([dlb2.astype(f32), jnp.zeros((12,), f32)])
    (y5,) = _tc_call(s4, [32], p[0], p[1], y4, dinv, bias(gb4p),
                     lW1p, bias(lb1), lW2, bias(lb2),
                     dlW1, bias(dlb1), dlW2p, bias(dlb2p))

    p = prop(y5, 32)

    def s5(p0, p1, y, dv, W, b, y6_o):
        t = dv[...] * (p0[...] + p1[...] + y[...])
        y6_o[...] = dv[...] * jax.nn.relu(_mm(t, W[...]) + b[...])

    dgW1p = jnp.concatenate([dgW1.astype(f32), jnp.zeros((12, 40), f32)], 0)
    (y6,) = _tc_call(s5, [40], p[0], p[1], y5, dinv, dgW1p, bias(dgb1))

    p = prop(y6, 40)

    def s6(p0, p1, y, dv, W, b, y7_o):
        t = dv[...] * (p0[...] + p1[...] + y[...])
        y7_o[...] = dv[...] * jax.nn.relu(_mm(t, W[...]) + b[...])

    (y7,) = _tc_call(s6, [80], p[0], p[1], y6, dinv, dgW2, bias(dgb2))

    p = prop(y7, 80)

    def s7(p0, p1, y, dv, W3_, b3_, W4_, y8_o):
        t = dv[...] * (p0[...] + p1[...] + y[...])
        h = jax.nn.relu(_mm(t, W3_[...]) + b3_[...])
        y8_o[...] = dv[...] * _mm(h, W4_[...])

    dgW4p = jnp.concatenate([dgW4.astype(f32), jnp.zeros((160, 5), f32)], 1)
    (y8,) = _tc_call(s7, [8], p[0], p[1], y7, dinv, dgW3, bias(dgb3), dgW4p)

    p = prop(y8, 8)
    dgb4p = jnp.concatenate([dgb4.astype(f32), jnp.zeros((5,), f32)])

    def s8(p0, p1, y, dv, b, out_o):
        out_o[...] = jnp.tanh(
            dv[...] * (p0[...] + p1[...] + y[...]) + b[...])

    (out,) = _tc_call(s8, [8], p[0], p[1], y8, dinv, bias(dgb4p))
    return out[:N, :3]


# speedup vs baseline: 29.7354x; 1.0111x over previous
import functools

import jax
import jax.numpy as jnp
from jax import lax
from jax.experimental import pallas as pl
from jax.experimental.pallas import tpu as pltpu
from jax.experimental.pallas import tpu_sc as plsc

N = 10000
NP = 10240
NC, NS = 2, 16
NW = NC * NS
CH = 128
BN = 1280


_CFG = {128: (128, 80, 20), 80: (392, 26, 13), 40: (1000, 10, 10),
        32: (1000, 10, 10), 8: (5000, 2, 2)}


@functools.partial(jax.jit, static_argnums=(4, 5, 6))
def _sc_propagate(y, srcr, dstr, zeros, T, QB, constant_rows=False):
    F = y.shape[1]
    BCH = srcr.shape[2]
    rows_per_tile = NP // NS
    NB = 1 if constant_rows else 2
    mesh = plsc.VectorSubcoreMesh(core_axis_name="c", subcore_axis_name="s")

    def body(y_hbm, srcr_hbm, dstr_hbm, zeros_hbm, out_hbm,
             src_v, dst_v, rows, sems, acc):
        c = lax.axis_index("c")
        s = lax.axis_index("s")
        wid = s * NC + c
        row0 = s * rows_per_tile
        pltpu.sync_copy(zeros_hbm.at[pl.ds(row0, rows_per_tile)],
                        acc.at[pl.ds(row0, rows_per_tile)])
        if constant_rows:
            pltpu.sync_copy(y_hbm.at[pl.ds(0, BCH)], rows.at[0])
        plsc.subcore_barrier()

        def outer(q, carry):
            if not constant_rows:
                pltpu.sync_copy(srcr_hbm.at[wid, pl.ds(q * QB, QB)], src_v)
            pltpu.sync_copy(dstr_hbm.at[wid, pl.ds(q * QB, QB)], dst_v)
            if constant_rows:
                for k in range(QB):
                    pltpu.sync_copy(rows.at[0], acc.at[dst_v.at[k]],
                                    add=True)
                return carry
            for b in range(NB - 1):
                pltpu.async_copy(y_hbm.at[src_v.at[b]], rows.at[b],
                                 sems.at[b])
            for k in range(QB):
                b = k % NB
                nxt = k + NB - 1
                if nxt < QB:
                    bi = nxt % NB
                    pltpu.async_copy(y_hbm.at[src_v.at[nxt]], rows.at[bi],
                                     sems.at[bi])
                pltpu.make_async_copy(y_hbm.at[pl.ds(0, BCH)], rows.at[b],
                                      sems.at[b]).wait()
                pltpu.sync_copy(rows.at[b], acc.at[dst_v.at[k]], add=True)
            return carry

        lax.fori_loop(0, T // QB, outer, 0, unroll=False)
        plsc.subcore_barrier()
        pltpu.sync_copy(acc.at[pl.ds(row0, rows_per_tile)],
                        out_hbm.at[c, pl.ds(row0, rows_per_tile)])

    scratch = [
        pltpu.VMEM((QB, BCH), jnp.int32),
        pltpu.VMEM((QB, BCH), jnp.int32),
        pltpu.VMEM((NB, BCH, F), jnp.float32),
        pltpu.SemaphoreType.DMA((NB,)),
        pltpu.VMEM_SHARED((NP, F), jnp.float32),
    ]
    run = pl.kernel(
        body,
        out_type=jax.ShapeDtypeStruct((NC, NP, F), jnp.float32),
        mesh=mesh,
        scratch_types=scratch,
        compiler_params=pltpu.CompilerParams(use_tc_tiling_on_sc=False),
    )
    return run(y, srcr, dstr, zeros)


def _tc_call(fn, out_widths, *args):
    in_specs = []
    for a in args:
        if a.ndim == 2 and a.shape[0] == NP:
            in_specs.append(
                pl.BlockSpec((BN, a.shape[1]), lambda i: (i, 0)))
        else:
            nd = a.ndim
            in_specs.append(
                pl.BlockSpec(a.shape, lambda i, _nd=nd: (0,) * _nd))
    out_specs = [pl.BlockSpec((BN, w), lambda i: (i, 0)) for w in out_widths]
    out_shape = [jax.ShapeDtypeStruct((NP, w), jnp.float32)
                 for w in out_widths]
    outs = pl.pallas_call(
        fn,
        grid=(NP // BN,),
        in_specs=in_specs,
        out_specs=out_specs,
        out_shape=out_shape,
    )(*args)
    return outs


def _mm(a, w):
    return jnp.dot(a, w, preferred_element_type=jnp.float32)


def kernel(x, edge_index, gW1, gb1, gW2, gb2, gW3, gb3, gW4, gb4,
           lW1, lb1, lW2, lb2, dlW1, dlb1, dlW2, dlb2,
           dgW1, dgb1, dgW2, dgb2, dgW3, dgb3, dgW4, dgb4):
    f32 = jnp.float32
    E = edge_index.shape[1]
    idx_arrays = {}
    for bch, t, _qb in _CFG.values():
        if bch in idx_arrays:
            continue
        ep = NW * t * bch
        pad_idx = (N + jnp.arange(ep - E, dtype=jnp.int32) % 16)
        idx_arrays[bch] = (
            jnp.concatenate(
                [edge_index[0].astype(jnp.int32), pad_idx]).reshape(
                    NW, t, bch),
            jnp.concatenate(
                [edge_index[1].astype(jnp.int32), pad_idx]).reshape(
                    NW, t, bch))
    xp = jnp.concatenate([x.astype(f32), jnp.zeros((NP - N, 128), f32)])
    ones8 = jnp.ones((NP, 8), f32)
    z8 = jnp.zeros((NP, 8), f32)

    def bias(b):
        return b.reshape(1, -1).astype(f32)

    bch8, t8, qb8 = _CFG[8]
    dp = _sc_propagate(ones8, idx_arrays[bch8][0], idx_arrays[bch8][1],
                       z8, t8, qb8, True)

    def s0(dp0, dp1, x_r, dinv_o, y1_o):
        dinv = lax.rsqrt(dp0[:, 0:1] + dp1[:, 0:1] + 1.0)
        dinv_o[...] = dinv
        y1_o[...] = dinv * x_r[...]

    dinv, y1 = _tc_call(s0, [1, 128], dp[0], dp[1], xp)

    def prop(y, F):
        bch, t, qb = _CFG[F]
        zz = jnp.zeros((NP, F), f32)
        return _sc_propagate(y, idx_arrays[bch][0], idx_arrays[bch][1],
                             zz, t, qb, False)

    p = prop(y1, 128)

    def s1(p0, p1, y, dv, W1, b1, W2, y2_o):
        t = dv[...] * (p0[...] + p1[...] + y[...])
        h = jax.nn.relu(_mm(t, W1[...]) + b1[...])
        y2_o[...] = dv[...] * _mm(h, W2[...])

    (y2,) = _tc_call(s1, [80], p[0], p[1], y1, dinv, gW1, bias(gb1), gW2)

    p = prop(y2, 80)

    def s2(p0, p1, y, dv, b2, W3, y3_o):
        h = jax.nn.relu(dv[...] * (p0[...] + p1[...] + y[...]) + b2[...])
        y3_o[...] = dv[...] * _mm(h, W3[...])

    (y3,) = _tc_call(s2, [40], p[0], p[1], y2, dinv, bias(gb2), gW3)

    p = prop(y3, 40)

    def s3(p0, p1, y, dv, b3, W4, y4_o):
        h = jax.nn.relu(dv[...] * (p0[...] + p1[...] + y[...]) + b3[...])
        y4_o[...] = dv[...] * _mm(h, W4[...])

    gW4p = jnp.concatenate([gW4.astype(f32), jnp.zeros((40, 12), f32)], 1)
    (y4,) = _tc_call(s3, [32], p[0], p[1], y3, dinv, bias(gb3), gW4p)

    p = prop(y4, 32)

    def s4(p0, p1, y, dv, b4, W_l1, b_l1, W_l2, b_l2,
           W_d1, b_d1, W_d2, b_d2, y5_o):
        h = jax.nn.relu(dv[...] * (p0[...] + p1[...] + y[...]) + b4[...])
        h = jax.nn.relu(_mm(h, W_l1[...]) + b_l1[...])
        h = _mm(h, W_l2[...]) + b_l2[...]
        h = jax.nn.relu(_mm(h, W_d1[...]) + b_d1[...])
        h = jax.nn.relu(_mm(h, W_d2[...]) + b_d2[...])
        y5_o[...] = dv[...] * h

    gb4p = jnp.concatenate([gb4.astype(f32), jnp.zeros((12,), f32)])
    lW1p = jnp.concatenate([lW1.astype(f32), jnp.zeros((12, 10), f32)], 0)
    dlW2p = jnp.concatenate([dlW2.astype(f32), jnp.zeros((10, 12), f32)], 1)
    dlb2p = jnp.concatenate([dlb2.astype(f32), jnp.zeros((12,), f32)])
    (y5,) = _tc_call(s4, [32], p[0], p[1], y4, dinv, bias(gb4p),
                     lW1p, bias(lb1), lW2, bias(lb2),
                     dlW1, bias(dlb1), dlW2p, bias(dlb2p))

    p = prop(y5, 32)

    def s5(p0, p1, y, dv, W, b, y6_o):
        t = dv[...] * (p0[...] + p1[...] + y[...])
        y6_o[...] = dv[...] * jax.nn.relu(_mm(t, W[...]) + b[...])

    dgW1p = jnp.concatenate([dgW1.astype(f32), jnp.zeros((12, 40), f32)], 0)
    (y6,) = _tc_call(s5, [40], p[0], p[1], y5, dinv, dgW1p, bias(dgb1))

    p = prop(y6, 40)

    def s6(p0, p1, y, dv, W, b, y7_o):
        t = dv[...] * (p0[...] + p1[...] + y[...])
        y7_o[...] = dv[...] * jax.nn.relu(_mm(t, W[...]) + b[...])

    (y7,) = _tc_call(s6, [80], p[0], p[1], y6, dinv, dgW2, bias(dgb2))

    p = prop(y7, 80)

    def s7(p0, p1, y, dv, W3_, b3_, W4_, y8_o):
        t = dv[...] * (p0[...] + p1[...] + y[...])
        h = jax.nn.relu(_mm(t, W3_[...]) + b3_[...])
        y8_o[...] = dv[...] * _mm(h, W4_[...])

    dgW4p = jnp.concatenate([dgW4.astype(f32), jnp.zeros((160, 5), f32)], 1)
    (y8,) = _tc_call(s7, [8], p[0], p[1], y7, dinv, dgW3, bias(dgb3), dgW4p)

    p = prop(y8, 8)
    dgb4p = jnp.concatenate([dgb4.astype(f32), jnp.zeros((5,), f32)])

    def s8(p0, p1, y, dv, b, out_o):
        out_o[...] = jnp.tanh(
            dv[...] * (p0[...] + p1[...] + y[...]) + b[...])

    (out,) = _tc_call(s8, [8], p[0], p[1], y8, dinv, bias(dgb4p))
    return out[:N, :3]


# speedup vs baseline: 29.7591x; 1.0008x over previous
import functools

import jax
import jax.numpy as jnp
from jax import lax
from jax.experimental import pallas as pl
from jax.experimental.pallas import tpu as pltpu
from jax.experimental.pallas import tpu_sc as plsc

N = 10000
NP = 10240
NC, NS = 2, 16
NW = NC * NS
BN = 1280


_CFG = {128: (128, 80, 20), 80: (392, 26, 13), 40: (1000, 10, 10),
        32: (1000, 10, 10), 8: (5000, 2, 2)}


@functools.partial(jax.jit, static_argnums=(4, 5, 6))
def _sc_propagate(y, srcr, dstr, zeros, T, QB, constant_rows=False):
    F = y.shape[1]
    BCH = srcr.shape[2]
    rows_per_tile = NP // NS
    NB = 1 if constant_rows else 2
    mesh = plsc.VectorSubcoreMesh(core_axis_name="c", subcore_axis_name="s")

    def body(y_hbm, srcr_hbm, dstr_hbm, zeros_hbm, out_hbm,
             src_v, dst_v, rows, sems, acc):
        c = lax.axis_index("c")
        s = lax.axis_index("s")
        wid = s * NC + c
        row0 = s * rows_per_tile
        pltpu.sync_copy(zeros_hbm.at[pl.ds(row0, rows_per_tile)],
                        acc.at[pl.ds(row0, rows_per_tile)])
        if constant_rows:
            pltpu.sync_copy(y_hbm.at[pl.ds(0, BCH)], rows.at[0])
        plsc.subcore_barrier()

        def outer(q, carry):
            if not constant_rows:
                pltpu.sync_copy(srcr_hbm.at[wid, pl.ds(q * QB, QB)], src_v)
            pltpu.sync_copy(dstr_hbm.at[wid, pl.ds(q * QB, QB)], dst_v)
            if constant_rows:
                for k in range(QB):
                    pltpu.sync_copy(rows.at[0], acc.at[dst_v.at[k]],
                                    add=True)
                return carry
            for b in range(NB - 1):
                pltpu.async_copy(y_hbm.at[src_v.at[b]], rows.at[b],
                                 sems.at[b])
            for k in range(QB):
                b = k % NB
                nxt = k + NB - 1
                if nxt < QB:
                    bi = nxt % NB
                    pltpu.async_copy(y_hbm.at[src_v.at[nxt]], rows.at[bi],
                                     sems.at[bi])
                pltpu.make_async_copy(y_hbm.at[pl.ds(0, BCH)], rows.at[b],
                                      sems.at[b]).wait()
                pltpu.sync_copy(rows.at[b], acc.at[dst_v.at[k]], add=True)
            return carry

        lax.fori_loop(0, T // QB, outer, 0, unroll=False)
        plsc.subcore_barrier()
        pltpu.sync_copy(acc.at[pl.ds(row0, rows_per_tile)],
                        out_hbm.at[c, pl.ds(row0, rows_per_tile)])

    scratch = [
        pltpu.VMEM((QB, BCH), jnp.int32),
        pltpu.VMEM((QB, BCH), jnp.int32),
        pltpu.VMEM((NB, BCH, F), jnp.float32),
        pltpu.SemaphoreType.DMA((NB,)),
        pltpu.VMEM_SHARED((NP, F), jnp.float32),
    ]
    run = pl.kernel(
        body,
        out_type=jax.ShapeDtypeStruct((NC, NP, F), jnp.float32),
        mesh=mesh,
        scratch_types=scratch,
        compiler_params=pltpu.CompilerParams(use_tc_tiling_on_sc=False),
    )
    return run(y, srcr, dstr, zeros)


def _tc_call(fn, out_widths, *args):
    in_specs = []
    for a in args:
        if a.ndim == 2 and a.shape[0] == NP:
            in_specs.append(
                pl.BlockSpec((BN, a.shape[1]), lambda i: (i, 0)))
        else:
            nd = a.ndim
            in_specs.append(
                pl.BlockSpec(a.shape, lambda i, _nd=nd: (0,) * _nd))
    out_specs = [pl.BlockSpec((BN, w), lambda i: (i, 0)) for w in out_widths]
    out_shape = [jax.ShapeDtypeStruct((NP, w), jnp.float32)
                 for w in out_widths]
    outs = pl.pallas_call(
        fn,
        grid=(NP // BN,),
        in_specs=in_specs,
        out_specs=out_specs,
        out_shape=out_shape,
    )(*args)
    return outs


def _mm(a, w):
    return jnp.dot(a, w, preferred_element_type=jnp.float32)


def kernel(x, edge_index, gW1, gb1, gW2, gb2, gW3, gb3, gW4, gb4,
           lW1, lb1, lW2, lb2, dlW1, dlb1, dlW2, dlb2,
           dgW1, dgb1, dgW2, dgb2, dgW3, dgb3, dgW4, dgb4):
    f32 = jnp.float32
    E = edge_index.shape[1]
    idx_arrays = {}
    for bch, t, _qb in _CFG.values():
        if bch in idx_arrays:
            continue
        ep = NW * t * bch
        pad_idx = (N + jnp.arange(ep - E, dtype=jnp.int32) % 16)
        idx_arrays[bch] = (
            jnp.concatenate(
                [edge_index[0].astype(jnp.int32), pad_idx]).reshape(
                    NW, t, bch),
            jnp.concatenate(
                [edge_index[1].astype(jnp.int32), pad_idx]).reshape(
                    NW, t, bch))
    xp = jnp.concatenate([x.astype(f32), jnp.zeros((NP - N, 128), f32)])
    ones8 = jnp.ones((NP, 8), f32)
    z8 = jnp.zeros((NP, 8), f32)

    def bias(b):
        return b.reshape(1, -1).astype(f32)

    bch8, t8, qb8 = _CFG[8]
    dp = _sc_propagate(ones8, idx_arrays[bch8][0], idx_arrays[bch8][1],
                       z8, t8, qb8, True)

    def s0(dp0, dp1, x_r, dinv_o, y1_o):
        dinv = lax.rsqrt(dp0[:, 0:1] + dp1[:, 0:1] + 1.0)
        dinv_o[...] = dinv
        y1_o[...] = dinv * x_r[...]

    dinv, y1 = _tc_call(s0, [1, 128], dp[0], dp[1], xp)

    def prop(y, F):
        bch, t, qb = _CFG[F]
        zz = jnp.zeros((NP, F), f32)
        return _sc_propagate(y, idx_arrays[bch][0], idx_arrays[bch][1],
                             zz, t, qb, False)

    p = prop(y1, 128)

    def s1(p0, p1, y, dv, W1, b1, W2, y2_o):
        t = dv[...] * (p0[...] + p1[...] + y[...])
        h = jax.nn.relu(_mm(t, W1[...]) + b1[...])
        y2_o[...] = dv[...] * _mm(h, W2[...])

    (y2,) = _tc_call(s1, [80], p[0], p[1], y1, dinv, gW1, bias(gb1), gW2)

    p = prop(y2, 80)

    def s2(p0, p1, y, dv, b2, W3, y3_o):
        h = jax.nn.relu(dv[...] * (p0[...] + p1[...] + y[...]) + b2[...])
        y3_o[...] = dv[...] * _mm(h, W3[...])

    (y3,) = _tc_call(s2, [40], p[0], p[1], y2, dinv, bias(gb2), gW3)

    p = prop(y3, 40)

    def s3(p0, p1, y, dv, b3, W4, y4_o):
        h = jax.nn.relu(dv[...] * (p0[...] + p1[...] + y[...]) + b3[...])
        y4_o[...] = dv[...] * _mm(h, W4[...])

    gW4p = jnp.concatenate([gW4.astype(f32), jnp.zeros((40, 12), f32)], 1)
    (y4,) = _tc_call(s3, [32], p[0], p[1], y3, dinv, bias(gb3), gW4p)

    p = prop(y4, 32)

    def s4(p0, p1, y, dv, b4, W_l1, b_l1, W_l2, b_l2,
           W_d1, b_d1, W_d2, b_d2, y5_o):
        h = jax.nn.relu(dv[...] * (p0[...] + p1[...] + y[...]) + b4[...])
        h = jax.nn.relu(_mm(h, W_l1[...]) + b_l1[...])
        h = _mm(h, W_l2[...]) + b_l2[...]
        h = jax.nn.relu(_mm(h, W_d1[...]) + b_d1[...])
        h = jax.nn.relu(_mm(h, W_d2[...]) + b_d2[...])
        y5_o[...] = dv[...] * h

    gb4p = jnp.concatenate([gb4.astype(f32), jnp.zeros((12,), f32)])
    lW1p = jnp.concatenate([lW1.astype(f32), jnp.zeros((12, 10), f32)], 0)
    dlW2p = jnp.concatenate([dlW2.astype(f32), jnp.zeros((10, 12), f32)], 1)
    dlb2p = jnp.concatenate([dlb2.astype(f32), jnp.zeros((12,), f32)])
    (y5,) = _tc_call(s4, [32], p[0], p[1], y4, dinv, bias(gb4p),
                     lW1p, bias(lb1), lW2, bias(lb2),
                     dlW1, bias(dlb1), dlW2p, bias(dlb2p))

    p = prop(y5, 32)

    def s5(p0, p1, y, dv, W, b, y6_o):
        t = dv[...] * (p0[...] + p1[...] + y[...])
        y6_o[...] = dv[...] * jax.nn.relu(_mm(t, W[...]) + b[...])

    dgW1p = jnp.concatenate([dgW1.astype(f32), jnp.zeros((12, 40), f32)], 0)
    (y6,) = _tc_call(s5, [40], p[0], p[1], y5, dinv, dgW1p, bias(dgb1))

    p = prop(y6, 40)

    def s6(p0, p1, y, dv, W, b, y7_o):
        t = dv[...] * (p0[...] + p1[...] + y[...])
        y7_o[...] = dv[...] * jax.nn.relu(_mm(t, W[...]) + b[...])

    (y7,) = _tc_call(s6, [80], p[0], p[1], y6, dinv, dgW2, bias(dgb2))

    p = prop(y7, 80)

    def s7(p0, p1, y, dv, W3_, b3_, W4_, y8_o):
        t = dv[...] * (p0[...] + p1[...] + y[...])
        h = jax.nn.relu(_mm(t, W3_[...]) + b3_[...])
        y8_o[...] = dv[...] * _mm(h, W4_[...])

    dgW4p = jnp.concatenate([dgW4.astype(f32), jnp.zeros((160, 5), f32)], 1)
    (y8,) = _tc_call(s7, [8], p[0], p[1], y7, dinv, dgW3, bias(dgb3), dgW4p)

    p = prop(y8, 8)
    dgb4p = jnp.concatenate([dgb4.astype(f32), jnp.zeros((5,), f32)])

    def s8(p0, p1, y, dv, b, out_o):
        out_o[...] = jnp.tanh(
            dv[...] * (p0[...] + p1[...] + y[...]) + b[...])

    (out,) = _tc_call(s8, [8], p[0], p[1], y8, dinv, bias(dgb4p))
    return out[:N, :3]
